# Initial kernel scaffold; baseline (speedup 1.0000x reference)
import functools

import jax
import jax.numpy as jnp
from jax import lax
from jax.experimental import pallas as pl
from jax.experimental.pallas import tpu as pltpu
from jax.experimental.pallas import tpu_sc as plsc

N_NODES = 10000
NP = 10016
E_PAD = 163840
ER = E_PAD // 128
CHUNK = 128
D = 256
DH = 128
NC, NS = 2, 16
RPT = NP // NS
BN = 256
GRID_N = (NP + BN - 1) // BN

_mesh = plsc.VectorSubcoreMesh(
    core_axis_name="c", subcore_axis_name="s", num_cores=NC, num_subcores=NS
)


@functools.partial(
    pl.kernel,
    out_type=[
        jax.ShapeDtypeStruct((NC, NP, 16), jnp.float32),
        jax.ShapeDtypeStruct((NC, NP, 16), jnp.float32),
    ],
    mesh=_mesh,
    scratch_types=[
        pltpu.VMEM((ER // (NC * NS), 128), jnp.int32),
        pltpu.VMEM((ER // (NC * NS), 128), jnp.int32),
        pltpu.VMEM((RPT, 16), jnp.float32),
        pltpu.VMEM((CHUNK, 16), jnp.float32),
    ],
)
def _deg_kernel(src_hbm, dst_hbm, sp_hbm, dp_hbm, sidx, didx, zb, ones):
    def scoped(shist, dhist):
        c = lax.axis_index("c")
        s = lax.axis_index("s")
        wid = s * NC + c
        rows_per_tile = ER // (NC * NS)

        z16 = jnp.zeros((16,), jnp.float32)
        o16 = jnp.ones((16,), jnp.float32)

        def fill_z(i, _):
            zb[i, :] = z16
            return 0

        lax.fori_loop(0, RPT, fill_z, 0)

        def fill_o(i, _):
            ones[i, :] = o16
            return 0

        lax.fori_loop(0, CHUNK, fill_o, 0)

        row0 = s * RPT
        pltpu.sync_copy(zb, shist.at[pl.ds(row0, RPT)])
        pltpu.sync_copy(zb, dhist.at[pl.ds(row0, RPT)])

        pltpu.sync_copy(src_hbm.at[pl.ds(wid * rows_per_tile, rows_per_tile)], sidx)
        pltpu.sync_copy(dst_hbm.at[pl.ds(wid * rows_per_tile, rows_per_tile)], didx)
        plsc.subcore_barrier()

        def body(j, _):
            pltpu.sync_copy(ones, shist.at[sidx.at[j]], add=True)
            pltpu.sync_copy(ones, dhist.at[didx.at[j]], add=True)
            return 0

        lax.fori_loop(0, rows_per_tile, body, 0)
        plsc.subcore_barrier()

        pltpu.sync_copy(shist.at[pl.ds(row0, RPT)], sp_hbm.at[c, pl.ds(row0, RPT)])
        pltpu.sync_copy(dhist.at[pl.ds(row0, RPT)], dp_hbm.at[c, pl.ds(row0, RPT)])

    pl.run_scoped(
        scoped,
        plsc.MemoryRef((NP, 16), jnp.float32, pltpu.VMEM_SHARED),
        plsc.MemoryRef((NP, 16), jnp.float32, pltpu.VMEM_SHARED),
    )


@functools.partial(
    pl.kernel,
    out_type=jax.ShapeDtypeStruct((NC, NP, DH), jnp.float32),
    mesh=_mesh,
    scratch_types=[
        pltpu.VMEM((ER // NS, 128), jnp.int32),
        pltpu.VMEM((ER // NS, 128), jnp.int32),
        pltpu.VMEM((RPT, DH), jnp.float32),
        pltpu.VMEM((CHUNK, DH), jnp.float32),
        pltpu.SemaphoreType.DMA,
    ],
)
def _agg_kernel(xs_hbm, srcc_hbm, dst_hbm, out_hbm, sidx, didx, zb, rows, sem):
    def scoped(acc):
        c = lax.axis_index("c")
        s = lax.axis_index("s")
        rows_per_tile = ER // NS

        z16 = jnp.zeros((16,), jnp.float32)

        def fill_z(i, _):
            for k in range(DH // 16):
                zb[i, pl.ds(k * 16, 16)] = z16
            return 0

        lax.fori_loop(0, RPT, fill_z, 0)
        row0 = s * RPT
        pltpu.sync_copy(zb, acc.at[pl.ds(row0, RPT)])

        pltpu.sync_copy(srcc_hbm.at[c, pl.ds(s * rows_per_tile, rows_per_tile)], sidx)
        pltpu.sync_copy(dst_hbm.at[pl.ds(s * rows_per_tile, rows_per_tile)], didx)
        plsc.subcore_barrier()

        def body(j, _):
            pltpu.async_copy(xs_hbm.at[sidx.at[j]], rows, sem).wait()
            pltpu.sync_copy(rows, acc.at[didx.at[j]], add=True)
            return 0

        lax.fori_loop(0, rows_per_tile, body, 0)
        plsc.subcore_barrier()

        pltpu.sync_copy(acc.at[pl.ds(row0, RPT)], out_hbm.at[c, pl.ds(row0, RPT)])

    pl.run_scoped(scoped, plsc.MemoryRef((NP, DH), jnp.float32, pltpu.VMEM_SHARED))


def _prep_body(x_ref, ds_ref, dd_ref, xs_ref, di_ref, do_ref):
    do = lax.rsqrt(jnp.maximum(ds_ref[...], 1.0))
    di = lax.rsqrt(jnp.maximum(dd_ref[...], 1.0))
    do_ref[...] = do
    di_ref[...] = di
    xs = x_ref[...] * do
    xs_ref[0] = xs[:, :DH]
    xs_ref[1] = xs[:, DH:]


def _prep_call(x, deg_s, deg_d):
    return pl.pallas_call(
        _prep_body,
        grid=(GRID_N,),
        in_specs=[
            pl.BlockSpec((BN, D), lambda i: (i, 0)),
            pl.BlockSpec((BN, 1), lambda i: (i, 0)),
            pl.BlockSpec((BN, 1), lambda i: (i, 0)),
        ],
        out_specs=[
            pl.BlockSpec((NC, BN, DH), lambda i: (0, i, 0)),
            pl.BlockSpec((BN, 1), lambda i: (i, 0)),
            pl.BlockSpec((BN, 1), lambda i: (i, 0)),
        ],
        out_shape=[
            jax.ShapeDtypeStruct((NC, NP, DH), jnp.float32),
            jax.ShapeDtypeStruct((NP, 1), jnp.float32),
            jax.ShapeDtypeStruct((NP, 1), jnp.float32),
        ],
    )(x, deg_s, deg_d)


def _mid_body(a_ref, di_ref, do_ref, w_ref, b_ref, o_ref):
    di = di_ref[...]
    r = (
        jnp.dot(a_ref[0] * di, w_ref[:DH, :], preferred_element_type=jnp.float32)
        + jnp.dot(a_ref[1] * di, w_ref[DH:, :], preferred_element_type=jnp.float32)
        + b_ref[...]
    )
    h = jnp.maximum(r, 0.0) * do_ref[...]
    o_ref[0] = h[:, :DH]
    o_ref[1] = h[:, DH:]


def _mid_call(agg, di, do, W, b):
    return pl.pallas_call(
        _mid_body,
        grid=(GRID_N,),
        in_specs=[
            pl.BlockSpec((NC, BN, DH), lambda i: (0, i, 0)),
            pl.BlockSpec((BN, 1), lambda i: (i, 0)),
            pl.BlockSpec((BN, 1), lambda i: (i, 0)),
            pl.BlockSpec((D, D), lambda i: (0, 0)),
            pl.BlockSpec((1, D), lambda i: (0, 0)),
        ],
        out_specs=pl.BlockSpec((NC, BN, DH), lambda i: (0, i, 0)),
        out_shape=jax.ShapeDtypeStruct((NC, NP, DH), jnp.float32),
    )(agg, di, do, W, b)


def _final_body(a_ref, di_ref, w_ref, b_ref, o_ref):
    di = di_ref[...]
    o_ref[...] = (
        jnp.dot(a_ref[0] * di, w_ref[:DH, :], preferred_element_type=jnp.float32)
        + jnp.dot(a_ref[1] * di, w_ref[DH:, :], preferred_element_type=jnp.float32)
        + b_ref[...]
    )


def _final_call(agg, di, W, b):
    return pl.pallas_call(
        _final_body,
        grid=(GRID_N,),
        in_specs=[
            pl.BlockSpec((NC, BN, DH), lambda i: (0, i, 0)),
            pl.BlockSpec((BN, 1), lambda i: (i, 0)),
            pl.BlockSpec((D, D), lambda i: (0, 0)),
            pl.BlockSpec((1, D), lambda i: (0, 0)),
        ],
        out_specs=pl.BlockSpec((BN, D), lambda i: (i, 0)),
        out_shape=jax.ShapeDtypeStruct((N_NODES, D), jnp.float32),
    )(agg, di, W, b)


def kernel(features, edge_index, W1, b1, W2, b2):
    src = edge_index[0].astype(jnp.int32)
    dst = edge_index[1].astype(jnp.int32)
    padn = E_PAD - src.shape[0]
    fill = jnp.full((padn,), N_NODES, jnp.int32)
    src2 = jnp.concatenate([src, fill]).reshape(ER, 128)
    dst2 = jnp.concatenate([dst, fill]).reshape(ER, 128)
    src_cat = jnp.stack([src2, src2 + NP])

    sp, dp = _deg_kernel(src2, dst2)
    deg_s = (sp[0, :, 0] + sp[1, :, 0]).reshape(NP, 1)
    deg_d = (dp[0, :, 0] + dp[1, :, 0]).reshape(NP, 1)

    xs, di, do = _prep_call(features, deg_s, deg_d)
    agg1 = _agg_kernel(xs.reshape(NC * NP, DH), src_cat, dst2)
    h1 = _mid_call(agg1, di, do, W1, b1.reshape(1, D))
    agg2 = _agg_kernel(h1.reshape(NC * NP, DH), src_cat, dst2)
    return _final_call(agg2, di, W2, b2.reshape(1, D))


# speedup vs baseline: 2.3690x; 2.3690x over previous
import functools

import jax
import jax.numpy as jnp
from jax import lax
from jax.experimental import pallas as pl
from jax.experimental.pallas import tpu as pltpu
from jax.experimental.pallas import tpu_sc as plsc

N_NODES = 10000
NP = 10112
E_PAD = 163840
ER = E_PAD // 128
CHUNK = 128
D = 256
DH = 128
DQ = 64
NQ = 4
NC, NS = 2, 16
RPT = NP // NS
BN = 256
GRID_N = (NP + BN - 1) // BN

_mesh = plsc.VectorSubcoreMesh(
    core_axis_name="c", subcore_axis_name="s", num_cores=NC, num_subcores=NS
)


@functools.partial(
    pl.kernel,
    out_type=[
        jax.ShapeDtypeStruct((NC, NP, 16), jnp.float32),
        jax.ShapeDtypeStruct((NC, NP, 16), jnp.float32),
    ],
    mesh=_mesh,
    scratch_types=[
        pltpu.VMEM((ER // (NC * NS), 128), jnp.int32),
        pltpu.VMEM((ER // (NC * NS), 128), jnp.int32),
        pltpu.VMEM((RPT, 16), jnp.float32),
        pltpu.VMEM((CHUNK, 16), jnp.float32),
        pltpu.VMEM_SHARED((NP, 16), jnp.float32),
        pltpu.VMEM_SHARED((NP, 16), jnp.float32),
    ],
    compiler_params=pltpu.CompilerParams(use_tc_tiling_on_sc=False),
)
def _deg_kernel(src_hbm, dst_hbm, sp_hbm, dp_hbm, sidx, didx, zb, ones, shist, dhist):
    if True:
        c = lax.axis_index("c")
        s = lax.axis_index("s")
        wid = s * NC + c
        rows_per_tile = ER // (NC * NS)

        z16 = jnp.zeros((16,), jnp.float32)
        o16 = jnp.ones((16,), jnp.float32)

        def fill_z(i, _):
            zb[i, :] = z16
            return 0

        lax.fori_loop(0, RPT, fill_z, 0)

        def fill_o(i, _):
            ones[i, :] = o16
            return 0

        lax.fori_loop(0, CHUNK, fill_o, 0)

        row0 = s * RPT
        pltpu.sync_copy(zb, shist.at[pl.ds(row0, RPT)])
        pltpu.sync_copy(zb, dhist.at[pl.ds(row0, RPT)])

        pltpu.sync_copy(src_hbm.at[pl.ds(wid * rows_per_tile, rows_per_tile)], sidx)
        pltpu.sync_copy(dst_hbm.at[pl.ds(wid * rows_per_tile, rows_per_tile)], didx)
        plsc.subcore_barrier()

        def body(j, _):
            pltpu.sync_copy(ones, shist.at[sidx.at[j]], add=True)
            pltpu.sync_copy(ones, dhist.at[didx.at[j]], add=True)
            return 0

        lax.fori_loop(0, rows_per_tile, body, 0)
        plsc.subcore_barrier()

        pltpu.sync_copy(shist.at[pl.ds(row0, RPT)], sp_hbm.at[c, pl.ds(row0, RPT)])
        pltpu.sync_copy(dhist.at[pl.ds(row0, RPT)], dp_hbm.at[c, pl.ds(row0, RPT)])


@functools.partial(
    pl.kernel,
    out_type=jax.ShapeDtypeStruct((NQ, NP, DQ), jnp.float32),
    mesh=_mesh,
    scratch_types=[
        pltpu.VMEM((ER // NS, 128), jnp.int32),
        pltpu.VMEM((ER // NS, 128), jnp.int32),
        pltpu.VMEM((RPT, DQ), jnp.float32),
        pltpu.VMEM((CHUNK, DQ), jnp.float32),
        pltpu.SemaphoreType.DMA,
        pltpu.VMEM_SHARED((NP, DQ), jnp.float32),
    ],
    compiler_params=pltpu.CompilerParams(use_tc_tiling_on_sc=False),
)
def _agg_kernel(xs_hbm, srcc_hbm, dst_hbm, out_hbm, sidx, didx, zb, rows, sem, acc):
    c = lax.axis_index("c")
    s = lax.axis_index("s")
    rows_per_tile = ER // NS
    row0 = s * RPT

    z16 = jnp.zeros((16,), jnp.float32)

    def fill_z(i, _):
        for k in range(DQ // 16):
            zb[i, pl.ds(k * 16, 16)] = z16
        return 0

    lax.fori_loop(0, RPT, fill_z, 0)

    pltpu.sync_copy(dst_hbm.at[pl.ds(s * rows_per_tile, rows_per_tile)], didx)

    for p in range(2):
        q = c * 2 + p
        pltpu.sync_copy(zb, acc.at[pl.ds(row0, RPT)])
        pltpu.sync_copy(
            srcc_hbm.at[q, pl.ds(s * rows_per_tile, rows_per_tile)], sidx
        )
        plsc.subcore_barrier()

        def body(j, _):
            pltpu.async_copy(xs_hbm.at[sidx.at[j]], rows, sem).wait()
            pltpu.sync_copy(rows, acc.at[didx.at[j]], add=True)
            return 0

        lax.fori_loop(0, rows_per_tile, body, 0)
        plsc.subcore_barrier()

        pltpu.sync_copy(acc.at[pl.ds(row0, RPT)], out_hbm.at[q, pl.ds(row0, RPT)])


def _prep_body(x_ref, ds_ref, dd_ref, xs_ref, di_ref, do_ref):
    do = lax.rsqrt(jnp.maximum(ds_ref[...], 1.0))
    di = lax.rsqrt(jnp.maximum(dd_ref[...], 1.0))
    do_ref[...] = do
    di_ref[...] = di
    xs = x_ref[...] * do
    for q in range(NQ):
        xs_ref[q] = xs[:, q * DQ:(q + 1) * DQ]


def _prep_call(x, deg_s, deg_d):
    return pl.pallas_call(
        _prep_body,
        grid=(GRID_N,),
        in_specs=[
            pl.BlockSpec((BN, D), lambda i: (i, 0)),
            pl.BlockSpec((BN, 1), lambda i: (i, 0)),
            pl.BlockSpec((BN, 1), lambda i: (i, 0)),
        ],
        out_specs=[
            pl.BlockSpec((NQ, BN, DQ), lambda i: (0, i, 0)),
            pl.BlockSpec((BN, 1), lambda i: (i, 0)),
            pl.BlockSpec((BN, 1), lambda i: (i, 0)),
        ],
        out_shape=[
            jax.ShapeDtypeStruct((NQ, NP, DQ), jnp.float32),
            jax.ShapeDtypeStruct((NP, 1), jnp.float32),
            jax.ShapeDtypeStruct((NP, 1), jnp.float32),
        ],
    )(x, deg_s, deg_d)


def _mid_body(a_ref, di_ref, do_ref, w_ref, b_ref, o_ref):
    di = di_ref[...]
    r = b_ref[...].astype(jnp.float32)
    for q in range(NQ):
        r = r + jnp.dot(
            a_ref[q] * di,
            w_ref[q * DQ:(q + 1) * DQ, :],
            preferred_element_type=jnp.float32,
        )
    h = jnp.maximum(r, 0.0) * do_ref[...]
    for q in range(NQ):
        o_ref[q] = h[:, q * DQ:(q + 1) * DQ]


def _mid_call(agg, di, do, W, b):
    return pl.pallas_call(
        _mid_body,
        grid=(GRID_N,),
        in_specs=[
            pl.BlockSpec((NQ, BN, DQ), lambda i: (0, i, 0)),
            pl.BlockSpec((BN, 1), lambda i: (i, 0)),
            pl.BlockSpec((BN, 1), lambda i: (i, 0)),
            pl.BlockSpec((D, D), lambda i: (0, 0)),
            pl.BlockSpec((1, D), lambda i: (0, 0)),
        ],
        out_specs=pl.BlockSpec((NQ, BN, DQ), lambda i: (0, i, 0)),
        out_shape=jax.ShapeDtypeStruct((NQ, NP, DQ), jnp.float32),
    )(agg, di, do, W, b)


def _final_body(a_ref, di_ref, w_ref, b_ref, o_ref):
    di = di_ref[...]
    r = b_ref[...].astype(jnp.float32)
    for q in range(NQ):
        r = r + jnp.dot(
            a_ref[q] * di,
            w_ref[q * DQ:(q + 1) * DQ, :],
            preferred_element_type=jnp.float32,
        )
    o_ref[...] = r


def _final_call(agg, di, W, b):
    return pl.pallas_call(
        _final_body,
        grid=(GRID_N,),
        in_specs=[
            pl.BlockSpec((NQ, BN, DQ), lambda i: (0, i, 0)),
            pl.BlockSpec((BN, 1), lambda i: (i, 0)),
            pl.BlockSpec((D, D), lambda i: (0, 0)),
            pl.BlockSpec((1, D), lambda i: (0, 0)),
        ],
        out_specs=pl.BlockSpec((BN, D), lambda i: (i, 0)),
        out_shape=jax.ShapeDtypeStruct((N_NODES, D), jnp.float32),
    )(agg, di, W, b)


def kernel(features, edge_index, W1, b1, W2, b2):
    src = edge_index[0].astype(jnp.int32)
    dst = edge_index[1].astype(jnp.int32)
    padn = E_PAD - src.shape[0]
    fill = jnp.full((padn,), N_NODES, jnp.int32)
    src2 = jnp.concatenate([src, fill]).reshape(ER, 128)
    dst2 = jnp.concatenate([dst, fill]).reshape(ER, 128)
    src_cat = jnp.stack([src2 + q * NP for q in range(NQ)])

    sp, dp = _deg_kernel(src2, dst2)
    deg_s = (sp[0, :, 0] + sp[1, :, 0]).reshape(NP, 1)
    deg_d = (dp[0, :, 0] + dp[1, :, 0]).reshape(NP, 1)

    xs, di, do = _prep_call(features, deg_s, deg_d)
    agg1 = _agg_kernel(xs.reshape(NQ * NP, DQ), src_cat, dst2)
    h1 = _mid_call(agg1, di, do, W1, b1.reshape(1, D))
    agg2 = _agg_kernel(h1.reshape(NQ * NP, DQ), src_cat, dst2)
    return _final_call(agg2, di, W2, b2.reshape(1, D))


# speedup vs baseline: 2.9312x; 1.2373x over previous
import functools

import jax
import jax.numpy as jnp
from jax import lax
from jax.experimental import pallas as pl
from jax.experimental.pallas import tpu as pltpu
from jax.experimental.pallas import tpu_sc as plsc

N_NODES = 10000
NP = 10112
E_PAD = 163840
ER = E_PAD // 128
CHUNK = 128
D = 256
DH = 128
DQ = 64
NQ = 4
NC, NS = 2, 16
RPT = NP // NS
BN = 256
GRID_N = (NP + BN - 1) // BN

_mesh = plsc.VectorSubcoreMesh(
    core_axis_name="c", subcore_axis_name="s", num_cores=NC, num_subcores=NS
)


@functools.partial(
    pl.kernel,
    out_type=[
        jax.ShapeDtypeStruct((NC, NP, 16), jnp.float32),
        jax.ShapeDtypeStruct((NC, NP, 16), jnp.float32),
    ],
    mesh=_mesh,
    scratch_types=[
        pltpu.VMEM((ER // (NC * NS), 128), jnp.int32),
        pltpu.VMEM((ER // (NC * NS), 128), jnp.int32),
        pltpu.VMEM((RPT, 16), jnp.float32),
        pltpu.VMEM((CHUNK, 16), jnp.float32),
        pltpu.VMEM_SHARED((NP, 16), jnp.float32),
        pltpu.VMEM_SHARED((NP, 16), jnp.float32),
    ],
    compiler_params=pltpu.CompilerParams(use_tc_tiling_on_sc=False),
)
def _deg_kernel(src_hbm, dst_hbm, sp_hbm, dp_hbm, sidx, didx, zb, ones, shist, dhist):
    if True:
        c = lax.axis_index("c")
        s = lax.axis_index("s")
        wid = s * NC + c
        rows_per_tile = ER // (NC * NS)

        z16 = jnp.zeros((16,), jnp.float32)
        o16 = jnp.ones((16,), jnp.float32)

        def fill_z(i, _):
            zb[i, :] = z16
            return 0

        lax.fori_loop(0, RPT, fill_z, 0)

        def fill_o(i, _):
            ones[i, :] = o16
            return 0

        lax.fori_loop(0, CHUNK, fill_o, 0)

        row0 = s * RPT
        pltpu.sync_copy(zb, shist.at[pl.ds(row0, RPT)])
        pltpu.sync_copy(zb, dhist.at[pl.ds(row0, RPT)])

        pltpu.sync_copy(src_hbm.at[pl.ds(wid * rows_per_tile, rows_per_tile)], sidx)
        pltpu.sync_copy(dst_hbm.at[pl.ds(wid * rows_per_tile, rows_per_tile)], didx)
        plsc.subcore_barrier()

        def body(j, _):
            pltpu.sync_copy(ones, shist.at[sidx.at[j]], add=True)
            pltpu.sync_copy(ones, dhist.at[didx.at[j]], add=True)
            return 0

        lax.fori_loop(0, rows_per_tile, body, 0)
        plsc.subcore_barrier()

        pltpu.sync_copy(shist.at[pl.ds(row0, RPT)], sp_hbm.at[c, pl.ds(row0, RPT)])
        pltpu.sync_copy(dhist.at[pl.ds(row0, RPT)], dp_hbm.at[c, pl.ds(row0, RPT)])


@functools.partial(
    pl.kernel,
    out_type=jax.ShapeDtypeStruct((NQ, NP, DQ), jnp.float32),
    mesh=_mesh,
    scratch_types=[
        pltpu.VMEM((ER // NS, 128), jnp.int32),
        pltpu.VMEM((ER // NS, 128), jnp.int32),
        pltpu.VMEM((RPT, DQ), jnp.float32),
        pltpu.VMEM((CHUNK, DQ), jnp.float32),
        pltpu.VMEM((CHUNK, DQ), jnp.float32),
        pltpu.SemaphoreType.DMA,
        pltpu.SemaphoreType.DMA,
        pltpu.VMEM_SHARED((NP, DQ), jnp.float32),
    ],
    compiler_params=pltpu.CompilerParams(use_tc_tiling_on_sc=False),
)
def _agg_kernel(xs_hbm, srcc_hbm, dst_hbm, out_hbm, sidx, didx, zb, rows0, rows1,
                sem0, sem1, acc):
    c = lax.axis_index("c")
    s = lax.axis_index("s")
    rows_per_tile = ER // NS
    row0 = s * RPT

    z16 = jnp.zeros((16,), jnp.float32)

    def fill_z(i, _):
        for k in range(DQ // 16):
            zb[i, pl.ds(k * 16, 16)] = z16
        return 0

    lax.fori_loop(0, RPT, fill_z, 0)

    pltpu.sync_copy(dst_hbm.at[pl.ds(s * rows_per_tile, rows_per_tile)], didx)

    for p in range(2):
        q = c * 2 + p
        pltpu.sync_copy(zb, acc.at[pl.ds(row0, RPT)])
        pltpu.sync_copy(
            srcc_hbm.at[q, pl.ds(s * rows_per_tile, rows_per_tile)], sidx
        )
        plsc.subcore_barrier()

        pltpu.async_copy(xs_hbm.at[sidx.at[0]], rows0, sem0)

        def body(g, _):
            j = 2 * g
            pltpu.async_copy(xs_hbm.at[sidx.at[j + 1]], rows1, sem1)
            pltpu.make_async_copy(xs_hbm.at[sidx.at[j]], rows0, sem0).wait()
            pltpu.sync_copy(rows0, acc.at[didx.at[j]], add=True)

            @pl.when(g < rows_per_tile // 2 - 1)
            def _():
                pltpu.async_copy(xs_hbm.at[sidx.at[j + 2]], rows0, sem0)

            pltpu.make_async_copy(xs_hbm.at[sidx.at[j + 1]], rows1, sem1).wait()
            pltpu.sync_copy(rows1, acc.at[didx.at[j + 1]], add=True)
            return 0

        lax.fori_loop(0, rows_per_tile // 2, body, 0)
        plsc.subcore_barrier()

        pltpu.sync_copy(acc.at[pl.ds(row0, RPT)], out_hbm.at[q, pl.ds(row0, RPT)])


def _prep_body(x_ref, ds_ref, dd_ref, xs_ref, di_ref, do_ref):
    do = lax.rsqrt(jnp.maximum(ds_ref[...], 1.0))
    di = lax.rsqrt(jnp.maximum(dd_ref[...], 1.0))
    do_ref[...] = do
    di_ref[...] = di
    xs = x_ref[...] * do
    for q in range(NQ):
        xs_ref[q] = xs[:, q * DQ:(q + 1) * DQ]


def _prep_call(x, deg_s, deg_d):
    return pl.pallas_call(
        _prep_body,
        grid=(GRID_N,),
        in_specs=[
            pl.BlockSpec((BN, D), lambda i: (i, 0)),
            pl.BlockSpec((BN, 1), lambda i: (i, 0)),
            pl.BlockSpec((BN, 1), lambda i: (i, 0)),
        ],
        out_specs=[
            pl.BlockSpec((NQ, BN, DQ), lambda i: (0, i, 0)),
            pl.BlockSpec((BN, 1), lambda i: (i, 0)),
            pl.BlockSpec((BN, 1), lambda i: (i, 0)),
        ],
        out_shape=[
            jax.ShapeDtypeStruct((NQ, NP, DQ), jnp.float32),
            jax.ShapeDtypeStruct((NP, 1), jnp.float32),
            jax.ShapeDtypeStruct((NP, 1), jnp.float32),
        ],
    )(x, deg_s, deg_d)


def _mid_body(a_ref, di_ref, do_ref, w_ref, b_ref, o_ref):
    di = di_ref[...]
    r = b_ref[...].astype(jnp.float32)
    for q in range(NQ):
        r = r + jnp.dot(
            a_ref[q] * di,
            w_ref[q * DQ:(q + 1) * DQ, :],
            preferred_element_type=jnp.float32,
        )
    h = jnp.maximum(r, 0.0) * do_ref[...]
    for q in range(NQ):
        o_ref[q] = h[:, q * DQ:(q + 1) * DQ]


def _mid_call(agg, di, do, W, b):
    return pl.pallas_call(
        _mid_body,
        grid=(GRID_N,),
        in_specs=[
            pl.BlockSpec((NQ, BN, DQ), lambda i: (0, i, 0)),
            pl.BlockSpec((BN, 1), lambda i: (i, 0)),
            pl.BlockSpec((BN, 1), lambda i: (i, 0)),
            pl.BlockSpec((D, D), lambda i: (0, 0)),
            pl.BlockSpec((1, D), lambda i: (0, 0)),
        ],
        out_specs=pl.BlockSpec((NQ, BN, DQ), lambda i: (0, i, 0)),
        out_shape=jax.ShapeDtypeStruct((NQ, NP, DQ), jnp.float32),
    )(agg, di, do, W, b)


def _final_body(a_ref, di_ref, w_ref, b_ref, o_ref):
    di = di_ref[...]
    r = b_ref[...].astype(jnp.float32)
    for q in range(NQ):
        r = r + jnp.dot(
            a_ref[q] * di,
            w_ref[q * DQ:(q + 1) * DQ, :],
            preferred_element_type=jnp.float32,
        )
    o_ref[...] = r


def _final_call(agg, di, W, b):
    return pl.pallas_call(
        _final_body,
        grid=(GRID_N,),
        in_specs=[
            pl.BlockSpec((NQ, BN, DQ), lambda i: (0, i, 0)),
            pl.BlockSpec((BN, 1), lambda i: (i, 0)),
            pl.BlockSpec((D, D), lambda i: (0, 0)),
            pl.BlockSpec((1, D), lambda i: (0, 0)),
        ],
        out_specs=pl.BlockSpec((BN, D), lambda i: (i, 0)),
        out_shape=jax.ShapeDtypeStruct((N_NODES, D), jnp.float32),
    )(agg, di, W, b)


def kernel(features, edge_index, W1, b1, W2, b2):
    src = edge_index[0].astype(jnp.int32)
    dst = edge_index[1].astype(jnp.int32)
    padn = E_PAD - src.shape[0]
    fill = jnp.full((padn,), N_NODES, jnp.int32)
    src2 = jnp.concatenate([src, fill]).reshape(ER, 128)
    dst2 = jnp.concatenate([dst, fill]).reshape(ER, 128)
    src_cat = jnp.stack([src2 + q * NP for q in range(NQ)])

    sp, dp = _deg_kernel(src2, dst2)
    deg_s = (sp[0, :, 0] + sp[1, :, 0]).reshape(NP, 1)
    deg_d = (dp[0, :, 0] + dp[1, :, 0]).reshape(NP, 1)

    xs, di, do = _prep_call(features, deg_s, deg_d)
    agg1 = _agg_kernel(xs.reshape(NQ * NP, DQ), src_cat, dst2)
    h1 = _mid_call(agg1, di, do, W1, b1.reshape(1, D))
    agg2 = _agg_kernel(h1.reshape(NQ * NP, DQ), src_cat, dst2)
    return _final_call(agg2, di, W2, b2.reshape(1, D))
